# baseline (speedup 1.0000x reference)
import functools

import jax
import jax.numpy as jnp
from jax import lax
from jax.experimental import pallas as pl
from jax.experimental.pallas import tpu as pltpu
from jax.experimental.pallas import tpu_sc as plsc

N = 10000
E = 320000
H = 128
G = 64
NC = 2
NS = 16
EPT = E // (NC * NS)
K = 104
CC = EPT // K
KT = EPT - CC * K
RPT = 640
RLAST = N - (NS - 1) * RPT
NPAD = 10240
DCOL = NPAD // NS

_SC_MESH = plsc.VectorSubcoreMesh(core_axis_name="c", subcore_axis_name="s")


@functools.partial(
    pl.kernel,
    out_type=jax.ShapeDtypeStruct((NC, NPAD), jnp.float32),
    mesh=_SC_MESH,
    scratch_types=[
        pltpu.VMEM((EPT,), jnp.int32),
        pltpu.VMEM((NPAD,), jnp.float32),
        pltpu.VMEM((NS, DCOL), jnp.float32),
        pltpu.VMEM_SHARED((NS, NPAD), jnp.float32),
    ],
    compiler_params=pltpu.CompilerParams(needs_layout_passes=False),
)
def _deg_kernel(dst_hbm, deg_hbm, dst_v, cnt_v, tmp_v, stage_sh):
    c = lax.axis_index("c")
    s = lax.axis_index("s")
    pltpu.sync_copy(dst_hbm.at[c, s], dst_v)
    zero = jnp.zeros((16,), jnp.float32)

    def zbody(i, carry):
        cnt_v[pl.ds(i * 16, 16)] = zero
        return carry

    lax.fori_loop(0, NPAD // 16, zbody, 0)
    ones = jnp.full((16,), 1.0, jnp.float32)

    def cbody(t, carry):
        idx = dst_v[pl.ds(t * 16, 16)]
        plsc.addupdate_scatter(cnt_v, [idx], ones)
        return carry

    lax.fori_loop(0, EPT // 16, cbody, 0)
    pltpu.sync_copy(cnt_v, stage_sh.at[s])
    plsc.subcore_barrier()
    col0 = s * DCOL
    pltpu.sync_copy(stage_sh.at[:, pl.ds(col0, DCOL)], tmp_v)

    def rbody(k, carry):
        acc = tmp_v[0, pl.ds(k * 16, 16)]
        for r in range(1, NS):
            acc = acc + tmp_v[r, pl.ds(k * 16, 16)]
        cnt_v[pl.ds(k * 16, 16)] = acc
        return carry

    lax.fori_loop(0, DCOL // 16, rbody, 0)
    pltpu.sync_copy(cnt_v.at[pl.ds(0, DCOL)], deg_hbm.at[c, pl.ds(col0, DCOL)])


@functools.partial(
    pl.kernel,
    out_type=jax.ShapeDtypeStruct((NC, N, H), jnp.float32),
    mesh=_SC_MESH,
    scratch_types=[
        pltpu.VMEM((EPT,), jnp.int32),
        pltpu.VMEM((EPT,), jnp.int32),
        pltpu.VMEM((2, K, H), jnp.float32),
        pltpu.VMEM((KT, H), jnp.float32),
        pltpu.VMEM_SHARED((N, H), jnp.float32),
        pltpu.SemaphoreType.DMA((2,)),
    ],
    compiler_params=pltpu.CompilerParams(needs_layout_passes=False,
                                         use_tc_tiling_on_sc=False),
)
def _prop_kernel(g_hbm, z_hbm, ei_hbm, out_hbm,
                 src_v, dst_v, rows_v, tail_v, acc_sh, gsem):
    c = lax.axis_index("c")
    s = lax.axis_index("s")
    w = c * NS + s
    pltpu.sync_copy(ei_hbm.at[0, w], src_v)
    pltpu.sync_copy(ei_hbm.at[1, w], dst_v)
    base = s * RPT
    last = s == NS - 1

    @pl.when(jnp.logical_and(c == 0, ~last))
    def _():
        pltpu.sync_copy(g_hbm.at[pl.ds(base, RPT)], acc_sh.at[pl.ds(base, RPT)])

    @pl.when(jnp.logical_and(c == 0, last))
    def _():
        pltpu.sync_copy(g_hbm.at[pl.ds((NS - 1) * RPT, RLAST)],
                        acc_sh.at[pl.ds((NS - 1) * RPT, RLAST)])

    @pl.when(jnp.logical_and(c == 1, ~last))
    def _():
        pltpu.sync_copy(z_hbm.at[pl.ds(base, RPT)], acc_sh.at[pl.ds(base, RPT)])

    @pl.when(jnp.logical_and(c == 1, last))
    def _():
        pltpu.sync_copy(z_hbm.at[pl.ds((NS - 1) * RPT, RLAST)],
                        acc_sh.at[pl.ds((NS - 1) * RPT, RLAST)])

    plsc.subcore_barrier()

    def gath(j, b):
        pltpu.async_copy(g_hbm.at[src_v.at[pl.ds(j * K, K)]], rows_v.at[b],
                         gsem.at[b])

    def gwait(j, b):
        pltpu.make_async_copy(g_hbm.at[src_v.at[pl.ds(j * K, K)]],
                              rows_v.at[b], gsem.at[b]).wait()

    def scat(j, b):
        pltpu.sync_copy(rows_v.at[b], acc_sh.at[dst_v.at[pl.ds(j * K, K)]],
                        add=True)

    gath(0, 0)
    nbody = (CC - 1) // 2

    def body(i, carry):
        j = 2 * i
        gath(j + 1, 1)
        gwait(j, 0)
        scat(j, 0)
        gath(j + 2, 0)
        gwait(j + 1, 1)
        scat(j + 1, 1)
        return carry

    lax.fori_loop(0, nbody, body, 0)
    if CC - 2 * nbody == 2:
        gath(CC - 1, 1)
        gwait(CC - 2, 0)
        scat(CC - 2, 0)
        gwait(CC - 1, 1)
        scat(CC - 1, 1)
    else:
        gwait(CC - 1, 0)
        scat(CC - 1, 0)
    pltpu.sync_copy(g_hbm.at[src_v.at[pl.ds(CC * K, KT)]], tail_v)
    pltpu.sync_copy(tail_v, acc_sh.at[dst_v.at[pl.ds(CC * K, KT)]], add=True)
    plsc.subcore_barrier()

    @pl.when(s < NS - 1)
    def _():
        pltpu.sync_copy(acc_sh.at[pl.ds(base, RPT)],
                        out_hbm.at[c, pl.ds(base, RPT)])

    @pl.when(s == NS - 1)
    def _():
        pltpu.sync_copy(acc_sh.at[pl.ds((NS - 1) * RPT, RLAST)],
                        out_hbm.at[c, pl.ds((NS - 1) * RPT, RLAST)])


BN = 2000
NB = N // BN


def _prea_body(x_ref, w_ref, h_ref):
    h_ref[...] = jnp.dot(x_ref[...], w_ref[...],
                         preferred_element_type=jnp.float32)


def _preb_body(degp_ref, h_ref, dinv_ref, g_ref):
    deg = degp_ref[0] + degp_ref[1] + 1.0
    dinv = lax.rsqrt(jnp.maximum(deg, 1.0))
    dinv_ref[...] = dinv
    g_ref[...] = dinv * h_ref[...]


def _mid_body(p_ref, dinv_ref, b_ref, w_ref, o_ref):
    dv = dinv_ref[...]
    a = jnp.maximum(dv * (p_ref[0] + p_ref[1]) + b_ref[...], 0.0)
    o_ref[...] = jnp.dot(dv * a, w_ref[...],
                         preferred_element_type=jnp.float32)


def _final_body(p_ref, dinv_ref, b_ref, batch_ref, wl_ref, bl_ref, o_ref,
                sums_ref, cnt_ref):
    i = pl.program_id(0)

    @pl.when(i == 0)
    def _():
        sums_ref[...] = jnp.zeros_like(sums_ref)
        cnt_ref[...] = jnp.zeros_like(cnt_ref)

    h = dinv_ref[...] * (p_ref[0] + p_ref[1]) + b_ref[...]
    seg = lax.broadcasted_iota(jnp.int32, (BN, G), 1)
    pmat = (seg == batch_ref[...]).astype(jnp.float32)
    dn = (((0,), (0,)), ((), ()))
    sums_ref[...] += lax.dot_general(pmat, h, dn,
                                     preferred_element_type=jnp.float32)
    cnt_ref[...] += lax.dot_general(pmat, jnp.ones((BN, 1), jnp.float32), dn,
                                    preferred_element_type=jnp.float32)

    @pl.when(i == NB - 1)
    def _():
        pooled = sums_ref[...] / jnp.maximum(cnt_ref[...], 1.0)
        o_ref[...] = pooled @ wl_ref[...] + bl_ref[...]


def _row_spec(cols):
    return pl.BlockSpec((BN, cols), lambda i: (i, 0))


def _fix_spec(rows, cols):
    return pl.BlockSpec((rows, cols), lambda i: (0, 0))


def kernel(x, edge_index, batch, W1, b1, W2, b2, W3, b3, Wlin, blin):
    f32 = jnp.float32
    ei3 = edge_index.reshape(2, NC * NS, EPT)
    dst3 = edge_index[1].reshape(NC, NS, EPT)
    batch_col = batch.reshape(N, 1)
    zeros = jnp.zeros((N, H), f32)

    degp = _deg_kernel(dst3)
    degp_col = degp.reshape(NC, NPAD, 1)

    h1 = pl.pallas_call(
        _prea_body,
        grid=(NB,),
        in_specs=[_row_spec(H), _fix_spec(H, H)],
        out_specs=_row_spec(H),
        out_shape=jax.ShapeDtypeStruct((N, H), f32),
    )(x, W1)

    dinv, g1 = pl.pallas_call(
        _preb_body,
        grid=(NB,),
        in_specs=[pl.BlockSpec((NC, BN, 1), lambda i: (0, i, 0)),
                  _row_spec(H)],
        out_specs=(_row_spec(1), _row_spec(H)),
        out_shape=(jax.ShapeDtypeStruct((N, 1), f32),
                   jax.ShapeDtypeStruct((N, H), f32)),
    )(degp_col, h1)

    def mid(p, b, W):
        return pl.pallas_call(
            _mid_body,
            grid=(NB,),
            in_specs=[pl.BlockSpec((NC, BN, H), lambda i: (0, i, 0)),
                      _row_spec(1), _fix_spec(1, H), _fix_spec(H, H)],
            out_specs=_row_spec(H),
            out_shape=jax.ShapeDtypeStruct((N, H), f32),
        )(p, dinv, b.reshape(1, H), W)

    p1 = _prop_kernel(g1, zeros, ei3)
    g2 = mid(p1, b1, W2)
    p2 = _prop_kernel(g2, zeros, ei3)
    g3 = mid(p2, b2, W3)
    p3 = _prop_kernel(g3, zeros, ei3)

    C = Wlin.shape[1]
    out = pl.pallas_call(
        _final_body,
        grid=(NB,),
        in_specs=[pl.BlockSpec((NC, BN, H), lambda i: (0, i, 0)),
                  _row_spec(1), _fix_spec(1, H),
                  _row_spec(1),
                  _fix_spec(H, C), _fix_spec(1, C)],
        out_specs=_fix_spec(G, C),
        out_shape=jax.ShapeDtypeStruct((G, C), f32),
        scratch_shapes=[pltpu.VMEM((G, H), f32), pltpu.VMEM((G, 1), f32)],
    )(p3, dinv, b3.reshape(1, H), batch_col, Wlin, blin.reshape(1, C))
    return out


# speedup vs baseline: 1.0266x; 1.0266x over previous
import functools

import jax
import jax.numpy as jnp
from jax import lax
from jax.experimental import pallas as pl
from jax.experimental.pallas import tpu as pltpu
from jax.experimental.pallas import tpu_sc as plsc

N = 10000
E = 320000
H = 128
G = 64
NC = 2
NS = 16
EPT = E // (NC * NS)
K = 104
CC = EPT // K
KT = EPT - CC * K
RPT = 640
RLAST = N - (NS - 1) * RPT
NPAD = 10240
DCOL = NPAD // NS

_SC_MESH = plsc.VectorSubcoreMesh(core_axis_name="c", subcore_axis_name="s")


@functools.partial(
    pl.kernel,
    out_type=jax.ShapeDtypeStruct((NC, NPAD), jnp.float32),
    mesh=_SC_MESH,
    scratch_types=[
        pltpu.VMEM((EPT,), jnp.int32),
        pltpu.VMEM((NPAD,), jnp.float32),
        pltpu.VMEM((NS, DCOL), jnp.float32),
        pltpu.VMEM_SHARED((NS, NPAD), jnp.float32),
    ],
    compiler_params=pltpu.CompilerParams(needs_layout_passes=False),
)
def _deg_kernel(ei_hbm, deg_hbm, dst_v, cnt_v, tmp_v, stage_sh):
    c = lax.axis_index("c")
    s = lax.axis_index("s")
    w = c * NS + s
    pltpu.sync_copy(ei_hbm.at[1, w], dst_v)
    zero = jnp.zeros((16,), jnp.float32)

    def zbody(i, carry):
        cnt_v[pl.ds(i * 16, 16)] = zero
        return carry

    lax.fori_loop(0, NPAD // 16, zbody, 0)
    ones = jnp.full((16,), 1.0, jnp.float32)

    def cbody(t, carry):
        idx = dst_v[pl.ds(t * 16, 16)]
        plsc.addupdate_scatter(cnt_v, [idx], ones)
        return carry

    lax.fori_loop(0, EPT // 16, cbody, 0)
    pltpu.sync_copy(cnt_v, stage_sh.at[s])
    plsc.subcore_barrier()
    col0 = s * DCOL
    pltpu.sync_copy(stage_sh.at[:, pl.ds(col0, DCOL)], tmp_v)

    def rbody(k, carry):
        acc = tmp_v[0, pl.ds(k * 16, 16)]
        for r in range(1, NS):
            acc = acc + tmp_v[r, pl.ds(k * 16, 16)]
        cnt_v[pl.ds(k * 16, 16)] = acc
        return carry

    lax.fori_loop(0, DCOL // 16, rbody, 0)
    pltpu.sync_copy(cnt_v.at[pl.ds(0, DCOL)], deg_hbm.at[c, pl.ds(col0, DCOL)])


@functools.partial(
    pl.kernel,
    out_type=jax.ShapeDtypeStruct((NC, N, H), jnp.float32),
    mesh=_SC_MESH,
    scratch_types=[
        pltpu.VMEM((EPT,), jnp.int32),
        pltpu.VMEM((EPT,), jnp.int32),
        pltpu.VMEM((2, K, H), jnp.float32),
        pltpu.VMEM((KT, H), jnp.float32),
        pltpu.VMEM_SHARED((N, H), jnp.float32),
        pltpu.SemaphoreType.DMA((2,)),
    ],
    compiler_params=pltpu.CompilerParams(needs_layout_passes=False,
                                         use_tc_tiling_on_sc=False),
)
def _prop_kernel(g_hbm, z_hbm, ei_hbm, out_hbm,
                 src_v, dst_v, rows_v, tail_v, acc_sh, gsem):
    c = lax.axis_index("c")
    s = lax.axis_index("s")
    w = c * NS + s
    pltpu.sync_copy(ei_hbm.at[0, w], src_v)
    pltpu.sync_copy(ei_hbm.at[1, w], dst_v)
    base = s * RPT
    last = s == NS - 1

    @pl.when(jnp.logical_and(c == 0, ~last))
    def _():
        pltpu.sync_copy(g_hbm.at[pl.ds(base, RPT)], acc_sh.at[pl.ds(base, RPT)])

    @pl.when(jnp.logical_and(c == 0, last))
    def _():
        pltpu.sync_copy(g_hbm.at[pl.ds((NS - 1) * RPT, RLAST)],
                        acc_sh.at[pl.ds((NS - 1) * RPT, RLAST)])

    @pl.when(jnp.logical_and(c == 1, ~last))
    def _():
        pltpu.sync_copy(z_hbm.at[pl.ds(base, RPT)], acc_sh.at[pl.ds(base, RPT)])

    @pl.when(jnp.logical_and(c == 1, last))
    def _():
        pltpu.sync_copy(z_hbm.at[pl.ds((NS - 1) * RPT, RLAST)],
                        acc_sh.at[pl.ds((NS - 1) * RPT, RLAST)])

    plsc.subcore_barrier()

    def gath(j, b):
        pltpu.async_copy(g_hbm.at[src_v.at[pl.ds(j * K, K)]], rows_v.at[b],
                         gsem.at[b])

    def gwait(j, b):
        pltpu.make_async_copy(g_hbm.at[src_v.at[pl.ds(j * K, K)]],
                              rows_v.at[b], gsem.at[b]).wait()

    def scat(j, b):
        pltpu.sync_copy(rows_v.at[b], acc_sh.at[dst_v.at[pl.ds(j * K, K)]],
                        add=True)

    gath(0, 0)
    nbody = (CC - 1) // 2

    def body(i, carry):
        j = 2 * i
        gath(j + 1, 1)
        gwait(j, 0)
        scat(j, 0)
        gath(j + 2, 0)
        gwait(j + 1, 1)
        scat(j + 1, 1)
        return carry

    lax.fori_loop(0, nbody, body, 0)
    if CC - 2 * nbody == 2:
        gath(CC - 1, 1)
        gwait(CC - 2, 0)
        scat(CC - 2, 0)
        gwait(CC - 1, 1)
        scat(CC - 1, 1)
    else:
        gwait(CC - 1, 0)
        scat(CC - 1, 0)
    pltpu.sync_copy(g_hbm.at[src_v.at[pl.ds(CC * K, KT)]], tail_v)
    pltpu.sync_copy(tail_v, acc_sh.at[dst_v.at[pl.ds(CC * K, KT)]], add=True)
    plsc.subcore_barrier()

    @pl.when(s < NS - 1)
    def _():
        pltpu.sync_copy(acc_sh.at[pl.ds(base, RPT)],
                        out_hbm.at[c, pl.ds(base, RPT)])

    @pl.when(s == NS - 1)
    def _():
        pltpu.sync_copy(acc_sh.at[pl.ds((NS - 1) * RPT, RLAST)],
                        out_hbm.at[c, pl.ds((NS - 1) * RPT, RLAST)])


BN = 2000
NB = N // BN


def _pre_body(degp_ref, x_ref, w_ref, dinv_ref, g_ref):
    deg = degp_ref[0] + degp_ref[1] + 1.0
    dinv = lax.rsqrt(jnp.maximum(deg, 1.0))
    dinv_ref[...] = dinv
    g_ref[...] = dinv * jnp.dot(x_ref[...], w_ref[...],
                                preferred_element_type=jnp.float32)


def _mid_body(p_ref, dinv_ref, b_ref, w_ref, o_ref):
    dv = dinv_ref[...]
    a = jnp.maximum(dv * (p_ref[0] + p_ref[1]) + b_ref[...], 0.0)
    o_ref[...] = jnp.dot(dv * a, w_ref[...],
                         preferred_element_type=jnp.float32)


def _final_body(p_ref, dinv_ref, b_ref, batch_ref, wl_ref, bl_ref, o_ref,
                sums_ref, cnt_ref):
    i = pl.program_id(0)

    @pl.when(i == 0)
    def _():
        sums_ref[...] = jnp.zeros_like(sums_ref)
        cnt_ref[...] = jnp.zeros_like(cnt_ref)

    h = dinv_ref[...] * (p_ref[0] + p_ref[1]) + b_ref[...]
    seg = lax.broadcasted_iota(jnp.int32, (G, BN), 0)
    pmat = (seg == batch_ref[0]).astype(jnp.float32)
    sums_ref[...] += jnp.dot(pmat, h, preferred_element_type=jnp.float32)
    cnt_ref[...] += jnp.dot(pmat, jnp.ones((BN, 1), jnp.float32),
                            preferred_element_type=jnp.float32)

    @pl.when(i == NB - 1)
    def _():
        pooled = sums_ref[...] / jnp.maximum(cnt_ref[...], 1.0)
        o_ref[...] = pooled @ wl_ref[...] + bl_ref[...]


def _row_spec(cols):
    return pl.BlockSpec((BN, cols), lambda i: (i, 0))


def _fix_spec(rows, cols):
    return pl.BlockSpec((rows, cols), lambda i: (0, 0))


def kernel(x, edge_index, batch, W1, b1, W2, b2, W3, b3, Wlin, blin):
    f32 = jnp.float32
    ei3 = edge_index.reshape(2, NC * NS, EPT)
    batch5 = batch.reshape(NB, 1, BN)
    zeros = jnp.zeros((N, H), f32)

    degp = _deg_kernel(ei3)
    degp_col = degp.reshape(NC, NPAD, 1)

    dinv, g1 = pl.pallas_call(
        _pre_body,
        grid=(NB,),
        in_specs=[pl.BlockSpec((NC, BN, 1), lambda i: (0, i, 0)),
                  _row_spec(H), _fix_spec(H, H)],
        out_specs=(_row_spec(1), _row_spec(H)),
        out_shape=(jax.ShapeDtypeStruct((N, 1), f32),
                   jax.ShapeDtypeStruct((N, H), f32)),
    )(degp_col, x, W1)

    def mid(p, b, W):
        return pl.pallas_call(
            _mid_body,
            grid=(NB,),
            in_specs=[pl.BlockSpec((NC, BN, H), lambda i: (0, i, 0)),
                      _row_spec(1), _fix_spec(1, H), _fix_spec(H, H)],
            out_specs=_row_spec(H),
            out_shape=jax.ShapeDtypeStruct((N, H), f32),
        )(p, dinv, b.reshape(1, H), W)

    p1 = _prop_kernel(g1, zeros, ei3)
    g2 = mid(p1, b1, W2)
    p2 = _prop_kernel(g2, zeros, ei3)
    g3 = mid(p2, b2, W3)
    p3 = _prop_kernel(g3, zeros, ei3)

    C = Wlin.shape[1]
    out = pl.pallas_call(
        _final_body,
        grid=(NB,),
        in_specs=[pl.BlockSpec((NC, BN, H), lambda i: (0, i, 0)),
                  _row_spec(1), _fix_spec(1, H),
                  pl.BlockSpec((1, 1, BN), lambda i: (i, 0, 0)),
                  _fix_spec(H, C), _fix_spec(1, C)],
        out_specs=_fix_spec(G, C),
        out_shape=jax.ShapeDtypeStruct((G, C), f32),
        scratch_shapes=[pltpu.VMEM((G, H), f32), pltpu.VMEM((G, 1), f32)],
    )(p3, dinv, b3.reshape(1, H), batch5, Wlin, blin.reshape(1, C))
    return out
